# Initial kernel scaffold; baseline (speedup 1.0000x reference)
import jax
import jax.numpy as jnp
from jax.experimental import pallas as pl


def kernel(x, token_table, pos_table):
    raise NotImplementedError("write your pallas kernel here")


# speedup vs baseline: 2.9748x; 2.9748x over previous
import functools

import jax
import jax.numpy as jnp
from jax import lax
from jax.experimental import pallas as pl
from jax.experimental.pallas import tpu as pltpu
from jax.experimental.pallas import tpu_sc as plsc

MAXLEN = 200
VOCAB = 100000
EMBED_DIM = 64
BATCH = 1024

HALF = MAXLEN // 2
NW = 32
SEQ_PER_W = BATCH // NW
C = 4
NCHUNK = SEQ_PER_W // C
NLANE = EMBED_DIM // 16


def _make_kernel():
    mesh = plsc.VectorSubcoreMesh(core_axis_name="c", subcore_axis_name="s")

    @functools.partial(
        pl.kernel,
        out_type=jax.ShapeDtypeStruct((BATCH * 2, HALF, EMBED_DIM), jnp.float32),
        mesh=mesh,
        compiler_params=pltpu.CompilerParams(use_tc_tiling_on_sc=False),
        scratch_types=[
            pltpu.VMEM((2 * C, HALF), jnp.int32),
            pltpu.VMEM((2 * C, HALF, EMBED_DIM), jnp.float32),
            pltpu.VMEM((2, HALF, EMBED_DIM), jnp.float32),
            pltpu.SemaphoreType.DMA,
        ],
    )
    def tok_pos_kernel(x_hbm, tok_hbm, pos_hbm, out_hbm, idx_v, rows_v, pos_v, sem):
        wid = lax.axis_index("s") * 2 + lax.axis_index("c")

        pltpu.sync_copy(pos_hbm, pos_v)

        def chunk_body(ci, carry):
            h0 = (wid * SEQ_PER_W + ci * C) * 2
            pltpu.sync_copy(x_hbm.at[pl.ds(h0, 2 * C)], idx_v)

            copies = []
            for h in range(2 * C):
                copies.append(
                    pltpu.async_copy(tok_hbm.at[idx_v.at[h]], rows_v.at[h], sem)
                )
            for cp in copies:
                cp.wait()

            def add_body(u, carry2):
                for j in range(NLANE):
                    sl = pl.ds(j * 16, 16)
                    p0 = pos_v[0, u, sl]
                    p1 = pos_v[1, u, sl]
                    for h in range(2 * C):
                        p = p0 if h % 2 == 0 else p1
                        rows_v[h, u, sl] += p
                return carry2

            lax.fori_loop(0, HALF, add_body, 0)

            pltpu.sync_copy(rows_v, out_hbm.at[pl.ds(h0, 2 * C)])
            return carry

        lax.fori_loop(0, NCHUNK, chunk_body, 0)

    return tok_pos_kernel


_kernel = _make_kernel()


@jax.jit
def kernel(x, token_table, pos_table):
    x2 = x.astype(jnp.int32).reshape(BATCH * 2, HALF)
    pos2 = pos_table.reshape(2, HALF, EMBED_DIM)
    out = _kernel(x2, token_table, pos2)
    return out.reshape(BATCH, MAXLEN, EMBED_DIM)


# speedup vs baseline: 3.1826x; 1.0699x over previous
import functools

import jax
import jax.numpy as jnp
from jax import lax
from jax.experimental import pallas as pl
from jax.experimental.pallas import tpu as pltpu
from jax.experimental.pallas import tpu_sc as plsc

MAXLEN = 200
VOCAB = 100000
EMBED_DIM = 64
BATCH = 1024

HALF = MAXLEN // 2
NW = 32
SEQ_PER_W = BATCH // NW
C = 4
NCHUNK = SEQ_PER_W // C
NLANE = EMBED_DIM // 16


def _make_kernel():
    mesh = plsc.VectorSubcoreMesh(core_axis_name="c", subcore_axis_name="s")

    @functools.partial(
        pl.kernel,
        out_type=jax.ShapeDtypeStruct((BATCH * 2, HALF, EMBED_DIM), jnp.float32),
        mesh=mesh,
        compiler_params=pltpu.CompilerParams(use_tc_tiling_on_sc=False),
        scratch_types=[
            pltpu.VMEM((2 * C, HALF), jnp.int32),
            pltpu.VMEM((2 * C, HALF), jnp.int32),
            pltpu.VMEM((2 * C, HALF, EMBED_DIM), jnp.float32),
            pltpu.VMEM((2 * C, HALF, EMBED_DIM), jnp.float32),
            pltpu.VMEM((2, HALF, EMBED_DIM), jnp.float32),
            pltpu.SemaphoreType.DMA,
            pltpu.SemaphoreType.DMA,
            pltpu.SemaphoreType.DMA,
            pltpu.SemaphoreType.DMA,
        ],
    )
    def tok_pos_kernel(x_hbm, tok_hbm, pos_hbm, out_hbm,
                       idx_a, idx_b, rows_a, rows_b, pos_v,
                       sem_ga, sem_gb, sem_oa, sem_ob):
        wid = lax.axis_index("s") * 2 + lax.axis_index("c")
        base = wid * SEQ_PER_W * 2

        pltpu.sync_copy(pos_hbm, pos_v)

        bufs = [(idx_a, rows_a, sem_ga, sem_oa), (idx_b, rows_b, sem_gb, sem_ob)]

        def fire(ci, b):
            idx_v, rows_v, sem_g, _ = bufs[b]
            h0 = base + ci * (2 * C)
            pltpu.sync_copy(x_hbm.at[pl.ds(h0, 2 * C)], idx_v)
            return [
                pltpu.async_copy(tok_hbm.at[idx_v.at[h]], rows_v.at[h], sem_g)
                for h in range(2 * C)
            ]

        def add_pos(rows_v):
            def add_body(u, carry):
                for j in range(NLANE):
                    sl = pl.ds(j * 16, 16)
                    p0 = pos_v[0, u, sl]
                    p1 = pos_v[1, u, sl]
                    for h in range(2 * C):
                        rows_v[h, u, sl] += p0 if h % 2 == 0 else p1
                return carry

            lax.fori_loop(0, HALF, add_body, 0)

        g_handles = [None, None]
        out_handles = [None, None]
        g_handles[0] = fire(0, 0)
        for ci in range(NCHUNK):
            b = ci % 2
            nb = 1 - b
            if ci + 1 < NCHUNK:
                if out_handles[nb] is not None:
                    out_handles[nb].wait()
                g_handles[nb] = fire(ci + 1, nb)
            for h in g_handles[b]:
                h.wait()
            _, rows_v, _, sem_o = bufs[b]
            add_pos(rows_v)
            h0 = base + ci * (2 * C)
            out_handles[b] = pltpu.async_copy(
                rows_v, out_hbm.at[pl.ds(h0, 2 * C)], sem_o
            )
        out_handles[0].wait()
        out_handles[1].wait()

    return tok_pos_kernel


_kernel = _make_kernel()


@jax.jit
def kernel(x, token_table, pos_table):
    x2 = x.astype(jnp.int32).reshape(BATCH * 2, HALF)
    pos2 = pos_table.reshape(2, HALF, EMBED_DIM)
    out = _kernel(x2, token_table, pos2)
    return out.reshape(BATCH, MAXLEN, EMBED_DIM)
